# baseline (speedup 1.0000x reference)
import functools

import jax
import jax.numpy as jnp
from jax import lax
from jax.experimental import pallas as pl
from jax.experimental.pallas import tpu as pltpu
from jax.experimental.pallas import tpu_sc as plsc

NUM_CORES = 2
NUM_SUBCORES = 16
NW = NUM_CORES * NUM_SUBCORES

BLK = 640
SUB = 128
NSUB = BLK // SUB


@functools.partial(jax.jit, static_argnames=("n_per_w",))
def _gather_flat(idx_flat, table, n_per_w):
    d = table.shape[1]
    n_blk = n_per_w // BLK
    assert n_per_w % BLK == 0 and n_blk % 2 == 0 and n_blk >= 4
    mesh = plsc.VectorSubcoreMesh(
        core_axis_name="c", subcore_axis_name="s",
        num_cores=NUM_CORES, num_subcores=NUM_SUBCORES)

    @functools.partial(
        pl.kernel,
        out_type=jax.ShapeDtypeStruct((idx_flat.shape[0], d), jnp.float32),
        mesh=mesh,
        scratch_types=[
            pltpu.VMEM((BLK,), jnp.int32),
            pltpu.VMEM((BLK,), jnp.int32),
            pltpu.VMEM((BLK, d), jnp.float32),
            pltpu.VMEM((BLK, d), jnp.float32),
            pltpu.SemaphoreType.DMA,
            pltpu.SemaphoreType.DMA,
            pltpu.SemaphoreType.DMA,
            pltpu.SemaphoreType.DMA,
        ],
        compiler_params=pltpu.CompilerParams(use_tc_tiling_on_sc=False),
    )
    def k(table_hbm, idx_hbm, out_hbm, idx0, idx1, rows0, rows1,
          sem_g0, sem_g1, sem_s0, sem_s1):
        wid = lax.axis_index("s") * NUM_CORES + lax.axis_index("c")
        base = wid * n_per_w

        def fire_gather(g, idx_v, rows_v, sem):
            pltpu.sync_copy(idx_hbm.at[pl.ds(base + g * BLK, BLK)], idx_v)
            for j in range(NSUB):
                pltpu.async_copy(
                    table_hbm.at[idx_v.at[pl.ds(j * SUB, SUB)]],
                    rows_v.at[pl.ds(j * SUB, SUB)], sem)

        def wait_gather(idx_v, rows_v, sem):
            for j in range(NSUB):
                pltpu.make_async_copy(
                    table_hbm.at[idx_v.at[pl.ds(j * SUB, SUB)]],
                    rows_v.at[pl.ds(j * SUB, SUB)], sem).wait()

        def fire_store(g, rows_v, sem):
            pltpu.async_copy(rows_v, out_hbm.at[pl.ds(base + g * BLK, BLK)],
                             sem)

        def wait_store(g, rows_v, sem):
            pltpu.make_async_copy(
                rows_v, out_hbm.at[pl.ds(base + g * BLK, BLK)], sem).wait()

        fire_gather(0, idx0, rows0, sem_g0)
        fire_gather(1, idx1, rows1, sem_g1)
        wait_gather(idx0, rows0, sem_g0)
        fire_store(0, rows0, sem_s0)

        def step(kk, _):
            ga = 2 * kk + 1
            gb = 2 * kk + 2
            wait_store(ga - 1, rows0, sem_s0)
            fire_gather(gb, idx0, rows0, sem_g0)
            wait_gather(idx1, rows1, sem_g1)
            fire_store(ga, rows1, sem_s1)
            wait_store(gb - 1, rows1, sem_s1)
            fire_gather(gb + 1, idx1, rows1, sem_g1)
            wait_gather(idx0, rows0, sem_g0)
            fire_store(gb, rows0, sem_s0)
            return ()

        lax.fori_loop(0, (n_blk - 2) // 2, step, (), unroll=False)

        gl = n_blk - 1
        wait_store(gl - 1, rows0, sem_s0)
        wait_gather(idx1, rows1, sem_g1)
        fire_store(gl, rows1, sem_s1)
        wait_store(gl, rows1, sem_s1)

    return k(table, idx_flat)


def kernel(agent_types, table):
    b, h = agent_types.shape
    n = b * h
    idx_flat = agent_types.reshape(n).astype(jnp.int32)
    out = _gather_flat(idx_flat, table, n // NW)
    return out.reshape(b, h, table.shape[1])
